# Initial kernel scaffold; baseline (speedup 1.0000x reference)
import jax
import jax.numpy as jnp
from jax.experimental import pallas as pl


def kernel(query, item, memory_bank):
    raise NotImplementedError("write your pallas kernel here")


# speedup vs baseline: 1.0012x; 1.0012x over previous
import jax
import jax.numpy as jnp
from jax.experimental import pallas as pl

_EPS = 1e-12


def _pass_body(bank_ref, item_ref, query_ref, packed_ref, msum_ref):
    bank = bank_ref[0]
    itm = item_ref[0]
    qry = query_ref[0]
    K = bank.shape[0]
    D = bank.shape[1]

    inv_i = 1.0 / jnp.maximum(jnp.sqrt(jnp.sum(itm * itm, axis=1, keepdims=True)), _EPS)
    inv_q = 1.0 / jnp.maximum(jnp.sqrt(jnp.sum(qry * qry, axis=1, keepdims=True)), _EPS)

    nsq = jnp.sum(bank * bank, axis=1, keepdims=True)
    inv_b = 1.0 / jnp.maximum(jnp.sqrt(nsq), _EPS)
    d_i = jax.lax.dot_general(bank, itm, (((1,), (1,)), ((), ())),
                              preferred_element_type=jnp.float32)
    d_q = jax.lax.dot_general(bank, qry, (((1,), (1,)), ((), ())),
                              preferred_element_type=jnp.float32)
    s_i = d_i * inv_b * inv_i
    s_q = d_q * inv_b * inv_q

    e = jnp.exp(s_q)
    S = jnp.sum(e, axis=0, keepdims=True)
    R = jnp.sum(e * bank, axis=0, keepdims=True)

    m = jnp.max(s_i, axis=0, keepdims=True)
    kio = jax.lax.broadcasted_iota(jnp.int32, (K, 1), 0)
    idx = jnp.min(jnp.where(s_i >= m, kio, K), axis=0, keepdims=True)
    oh = (kio == idx).astype(jnp.float32)
    row_best = jnp.sum(oh * bank, axis=0, keepdims=True)
    sq_best = jnp.sum(oh * s_q, axis=0, keepdims=True)
    row0 = bank[0:1, :]
    sq_0 = s_q[0:1, :]

    e_best = jnp.exp(sq_best)
    e_0 = jnp.exp(sq_0)
    new_c = 0.5 * row_best + 0.5 * itm
    inv_nc = 1.0 / jnp.maximum(jnp.sqrt(jnp.sum(new_c * new_c, axis=1, keepdims=True)), _EPS)
    e_new_c = jnp.exp(jnp.sum(new_c * qry, axis=1, keepdims=True) * inv_nc * inv_q)
    e_new_o = jnp.exp(jnp.sum(itm * qry, axis=1, keepdims=True) * inv_i * inv_q)

    A_c = e_new_c * new_c - e_best * row_best
    A_o = e_new_o * itm - e_0 * row0
    dS_c = e_new_c - e_best
    dS_o = e_new_o - e_0

    lane = jax.lax.broadcasted_iota(jnp.int32, (1, D), 1)
    scal = (jnp.where(lane == 0, S, 0.0)
            + jnp.where(lane == 1, dS_c, 0.0)
            + jnp.where(lane == 2, dS_o, 0.0)
            + jnp.where(lane == 3, m, 0.0))

    packed_ref[0, 0:1, :] = R
    packed_ref[0, 1:2, :] = A_c
    packed_ref[0, 2:3, :] = A_o
    packed_ref[0, 3:4, :] = scal

    b = pl.program_id(0)

    @pl.when(b == 0)
    def _init():
        msum_ref[...] = jnp.zeros_like(msum_ref)

    msum_ref[...] += m


def _finalize_body(packed_ref, msum_ref, out_ref):
    pk = packed_ref[...]
    B = pk.shape[0]
    R = pk[:, 0, :]
    A_c = pk[:, 1, :]
    A_o = pk[:, 2, :]
    S = pk[:, 3, 0:1]
    dS_c = pk[:, 3, 1:2]
    dS_o = pk[:, 3, 2:3]
    cond = (msum_ref[0, 0] * (1.0 / B)) >= 0.5
    S_fin = S + jnp.where(cond, dS_c, dS_o)
    R_fin = R + jnp.where(cond, A_c, A_o)
    out_ref[...] = R_fin / S_fin


def kernel(query, item, memory_bank):
    B, K, D = memory_bank.shape
    q3 = query.reshape(B, 1, D)
    i3 = item.reshape(B, 1, D)

    packed, msum = pl.pallas_call(
        _pass_body,
        grid=(B,),
        in_specs=[
            pl.BlockSpec((1, K, D), lambda b: (b, 0, 0)),
            pl.BlockSpec((1, 1, D), lambda b: (b, 0, 0)),
            pl.BlockSpec((1, 1, D), lambda b: (b, 0, 0)),
        ],
        out_specs=[
            pl.BlockSpec((1, 4, D), lambda b: (b, 0, 0)),
            pl.BlockSpec((1, 128), lambda b: (0, 0)),
        ],
        out_shape=[
            jax.ShapeDtypeStruct((B, 4, D), jnp.float32),
            jax.ShapeDtypeStruct((1, 128), jnp.float32),
        ],
    )(memory_bank, i3, q3)

    retrieved = pl.pallas_call(
        _finalize_body,
        out_shape=jax.ShapeDtypeStruct((B, D), jnp.float32),
    )(packed, msum)
    return retrieved


# speedup vs baseline: 1.0671x; 1.0658x over previous
import jax
import jax.numpy as jnp
from jax.experimental import pallas as pl

_EPS = 1e-12


def _pass_body(bank_ref, item_ref, query_ref, packed_ref, msum_ref):
    nb = bank_ref.shape[0]
    for s in range(nb):
        _one_stream(bank_ref, item_ref, query_ref, packed_ref, msum_ref, s)


def _one_stream(bank_ref, item_ref, query_ref, packed_ref, msum_ref, s):
    bank = bank_ref[s]
    itm = item_ref[s]
    qry = query_ref[s]
    K = bank.shape[0]
    D = bank.shape[1]

    inv_i = 1.0 / jnp.maximum(jnp.sqrt(jnp.sum(itm * itm, axis=1, keepdims=True)), _EPS)
    inv_q = 1.0 / jnp.maximum(jnp.sqrt(jnp.sum(qry * qry, axis=1, keepdims=True)), _EPS)

    nsq = jnp.sum(bank * bank, axis=1, keepdims=True)
    inv_b = 1.0 / jnp.maximum(jnp.sqrt(nsq), _EPS)
    d_i = jax.lax.dot_general(bank, itm, (((1,), (1,)), ((), ())),
                              preferred_element_type=jnp.float32)
    d_q = jax.lax.dot_general(bank, qry, (((1,), (1,)), ((), ())),
                              preferred_element_type=jnp.float32)
    s_i = d_i * inv_b * inv_i
    s_q = d_q * inv_b * inv_q

    e = jnp.exp(s_q)
    S = jnp.sum(e, axis=0, keepdims=True)
    R = jnp.sum(e * bank, axis=0, keepdims=True)

    m = jnp.max(s_i, axis=0, keepdims=True)
    kio = jax.lax.broadcasted_iota(jnp.int32, (K, 1), 0)
    idx = jnp.min(jnp.where(s_i >= m, kio, K), axis=0, keepdims=True)
    oh = (kio == idx).astype(jnp.float32)
    row_best = jnp.sum(oh * bank, axis=0, keepdims=True)
    sq_best = jnp.sum(oh * s_q, axis=0, keepdims=True)
    row0 = bank[0:1, :]
    sq_0 = s_q[0:1, :]

    e_best = jnp.exp(sq_best)
    e_0 = jnp.exp(sq_0)
    new_c = 0.5 * row_best + 0.5 * itm
    inv_nc = 1.0 / jnp.maximum(jnp.sqrt(jnp.sum(new_c * new_c, axis=1, keepdims=True)), _EPS)
    e_new_c = jnp.exp(jnp.sum(new_c * qry, axis=1, keepdims=True) * inv_nc * inv_q)
    e_new_o = jnp.exp(jnp.sum(itm * qry, axis=1, keepdims=True) * inv_i * inv_q)

    A_c = e_new_c * new_c - e_best * row_best
    A_o = e_new_o * itm - e_0 * row0
    dS_c = e_new_c - e_best
    dS_o = e_new_o - e_0

    lane = jax.lax.broadcasted_iota(jnp.int32, (1, D), 1)
    scal = (jnp.where(lane == 0, S, 0.0)
            + jnp.where(lane == 1, dS_c, 0.0)
            + jnp.where(lane == 2, dS_o, 0.0)
            + jnp.where(lane == 3, m, 0.0))

    packed_ref[s, 0:1, :] = R
    packed_ref[s, 1:2, :] = A_c
    packed_ref[s, 2:3, :] = A_o
    packed_ref[s, 3:4, :] = scal

    b = pl.program_id(0)

    @pl.when(jnp.logical_and(b == 0, s == 0))
    def _init():
        msum_ref[...] = jnp.zeros_like(msum_ref)

    msum_ref[...] += m


def _finalize_body(packed_ref, msum_ref, out_ref):
    pk = packed_ref[...]
    B = pk.shape[0]
    R = pk[:, 0, :]
    A_c = pk[:, 1, :]
    A_o = pk[:, 2, :]
    S = pk[:, 3, 0:1]
    dS_c = pk[:, 3, 1:2]
    dS_o = pk[:, 3, 2:3]
    cond = (msum_ref[0, 0] * (1.0 / B)) >= 0.5
    S_fin = S + jnp.where(cond, dS_c, dS_o)
    R_fin = R + jnp.where(cond, A_c, A_o)
    out_ref[...] = R_fin / S_fin


def kernel(query, item, memory_bank):
    B, K, D = memory_bank.shape
    q3 = query.reshape(B, 1, D)
    i3 = item.reshape(B, 1, D)

    BPB = 4
    packed, msum = pl.pallas_call(
        _pass_body,
        grid=(B // BPB,),
        in_specs=[
            pl.BlockSpec((BPB, K, D), lambda b: (b, 0, 0)),
            pl.BlockSpec((BPB, 1, D), lambda b: (b, 0, 0)),
            pl.BlockSpec((BPB, 1, D), lambda b: (b, 0, 0)),
        ],
        out_specs=[
            pl.BlockSpec((BPB, 4, D), lambda b: (b, 0, 0)),
            pl.BlockSpec((1, 128), lambda b: (0, 0)),
        ],
        out_shape=[
            jax.ShapeDtypeStruct((B, 4, D), jnp.float32),
            jax.ShapeDtypeStruct((1, 128), jnp.float32),
        ],
    )(memory_bank, i3, q3)

    retrieved = pl.pallas_call(
        _finalize_body,
        out_shape=jax.ShapeDtypeStruct((B, D), jnp.float32),
    )(packed, msum)
    return retrieved


# speedup vs baseline: 2.0770x; 1.9464x over previous
import jax
import jax.numpy as jnp
from jax.experimental import pallas as pl

_EPS = 1e-12


def _pass_body(bank_ref, ir_ref, qr_ref, packed_ref, msum_ref):
    nb, K, D = bank_ref.shape
    itm_all = ir_ref[:, 0, :]
    qry_all = qr_ref[:, 0, :]

    inv_i = 1.0 / jnp.maximum(jnp.sqrt(jnp.sum(itm_all * itm_all, axis=1, keepdims=True)), _EPS)
    inv_q = 1.0 / jnp.maximum(jnp.sqrt(jnp.sum(qry_all * qry_all, axis=1, keepdims=True)), _EPS)

    V = jnp.concatenate([itm_all, qry_all], axis=0)
    subn = jax.lax.broadcasted_iota(jnp.int32, (8, D), 0)
    ones_row = jnp.where(subn == 0, 1.0, 0.0)

    di_rows, dq_rows, ns_rows = [], [], []
    for s in range(nb):
        bank = bank_ref[s]
        MX = jax.lax.dot_general(V, bank, (((1,), (1,)), ((), ())),
                                 preferred_element_type=jnp.float32)
        NT = jax.lax.dot_general(ones_row, bank * bank, (((1,), (1,)), ((), ())),
                                 preferred_element_type=jnp.float32)
        di_rows.append(MX[s:s + 1, :])
        dq_rows.append(MX[nb + s:nb + s + 1, :])
        ns_rows.append(NT[0:1, :])

    d_i = jnp.concatenate(di_rows, axis=0)
    d_q = jnp.concatenate(dq_rows, axis=0)
    nsq = jnp.concatenate(ns_rows, axis=0)

    inv_b = 1.0 / jnp.maximum(jnp.sqrt(nsq), _EPS)
    s_i = d_i * inv_b * inv_i
    s_q = d_q * inv_b * inv_q

    e = jnp.exp(s_q)
    S = jnp.sum(e, axis=1, keepdims=True)

    m = jnp.max(s_i, axis=1, keepdims=True)
    kio = jax.lax.broadcasted_iota(jnp.int32, (nb, K), 1)
    idx = jnp.min(jnp.where(s_i >= m, kio, K), axis=1, keepdims=True)
    oh = (kio == idx).astype(jnp.float32)
    sq_best = jnp.sum(oh * s_q, axis=1, keepdims=True)
    sq_0 = s_q[:, 0:1]

    EO = jnp.concatenate([e, oh], axis=0)
    r_rows, rb_rows, r0_rows = [], [], []
    for s in range(nb):
        bank = bank_ref[s]
        R2 = jax.lax.dot_general(EO, bank, (((1,), (0,)), ((), ())),
                                 preferred_element_type=jnp.float32)
        r_rows.append(R2[s:s + 1, :])
        rb_rows.append(R2[nb + s:nb + s + 1, :])
        r0_rows.append(bank[0:1, :])

    R = jnp.concatenate(r_rows, axis=0)
    row_best = jnp.concatenate(rb_rows, axis=0)
    row0 = jnp.concatenate(r0_rows, axis=0)

    e_best = jnp.exp(sq_best)
    e_0 = jnp.exp(sq_0)
    new_c = 0.5 * row_best + 0.5 * itm_all
    inv_nc = 1.0 / jnp.maximum(jnp.sqrt(jnp.sum(new_c * new_c, axis=1, keepdims=True)), _EPS)
    e_new_c = jnp.exp(jnp.sum(new_c * qry_all, axis=1, keepdims=True) * inv_nc * inv_q)
    e_new_o = jnp.exp(jnp.sum(itm_all * qry_all, axis=1, keepdims=True) * inv_i * inv_q)

    A_c = e_new_c * new_c - e_best * row_best
    A_o = e_new_o * itm_all - e_0 * row0
    dS_c = e_new_c - e_best
    dS_o = e_new_o - e_0

    dlane = jax.lax.broadcasted_iota(jnp.int32, (nb, D), 1)
    scal = (jnp.where(dlane == 0, S, 0.0)
            + jnp.where(dlane == 1, dS_c, 0.0)
            + jnp.where(dlane == 2, dS_o, 0.0)
            + jnp.where(dlane == 3, m, 0.0))

    packed_ref[:, 0, :] = R
    packed_ref[:, 1, :] = A_c
    packed_ref[:, 2, :] = A_o
    packed_ref[:, 3, :] = scal

    b = pl.program_id(0)

    @pl.when(b == 0)
    def _init():
        msum_ref[...] = jnp.zeros_like(msum_ref)

    msum_ref[...] += jnp.sum(m)


def _finalize_body(packed_ref, msum_ref, out_ref):
    pk = packed_ref[...]
    B = pk.shape[0]
    R = pk[:, 0, :]
    A_c = pk[:, 1, :]
    A_o = pk[:, 2, :]
    S = pk[:, 3, 0:1]
    dS_c = pk[:, 3, 1:2]
    dS_o = pk[:, 3, 2:3]
    cond = (msum_ref[0, 0] * (1.0 / B)) >= 0.5
    S_fin = S + jnp.where(cond, dS_c, dS_o)
    R_fin = R + jnp.where(cond, A_c, A_o)
    out_ref[...] = R_fin / S_fin


def kernel(query, item, memory_bank):
    B, K, D = memory_bank.shape
    q3 = query.reshape(B, 1, D)
    i3 = item.reshape(B, 1, D)

    BPB = 8
    packed, msum = pl.pallas_call(
        _pass_body,
        grid=(B // BPB,),
        in_specs=[
            pl.BlockSpec((BPB, K, D), lambda b: (b, 0, 0)),
            pl.BlockSpec((BPB, 1, D), lambda b: (b, 0, 0)),
            pl.BlockSpec((BPB, 1, D), lambda b: (b, 0, 0)),
        ],
        out_specs=[
            pl.BlockSpec((BPB, 4, D), lambda b: (b, 0, 0)),
            pl.BlockSpec((1, 128), lambda b: (0, 0)),
        ],
        out_shape=[
            jax.ShapeDtypeStruct((B, 4, D), jnp.float32),
            jax.ShapeDtypeStruct((1, 128), jnp.float32),
        ],
    )(memory_bank, i3, q3)

    retrieved = pl.pallas_call(
        _finalize_body,
        out_shape=jax.ShapeDtypeStruct((B, D), jnp.float32),
    )(packed, msum)
    return retrieved
